# baseline (speedup 1.0000x reference)
import functools

import jax
import jax.numpy as jnp
from jax import lax
from jax.experimental import pallas as pl
from jax.experimental.pallas import tpu as pltpu
from jax.experimental.pallas import tpu_sc as plsc

VOCAB_SIZE = 100000
EMBED_DIM = 128

NC = 2
NS = 16
NW = NC * NS

IDX_W = 128
RPS = 2
NB = 3


def _make_gather(B):
    assert B % (NW * IDX_W) == 0
    n_rows = B // (NW * IDX_W)
    assert n_rows % RPS == 0
    n_super = n_rows // RPS
    n_loop = (n_super - NB) // NB
    n_tail = n_super - NB * n_loop
    assert n_loop >= 1 and NB <= n_tail <= 2 * NB

    mesh = plsc.VectorSubcoreMesh(core_axis_name="c", subcore_axis_name="s")

    @functools.partial(
        pl.kernel,
        mesh=mesh,
        out_type=jax.ShapeDtypeStruct((B // IDX_W, IDX_W, EMBED_DIM), jnp.float32),
        scratch_types=[
            pltpu.VMEM((n_rows, IDX_W), jnp.int32),
            pltpu.VMEM((NB, RPS, IDX_W, EMBED_DIM), jnp.float32),
            [pltpu.SemaphoreType.DMA] * NB,
            [pltpu.SemaphoreType.DMA] * NB,
        ],
    )
    def gather_kernel(table_hbm, idx_hbm, out_hbm, idx_v, rows_v, gsems, wsems):
        wid = lax.axis_index("s") * NC + lax.axis_index("c")
        out_base = wid * n_rows

        pltpu.sync_copy(idx_hbm.at[wid], idx_v)

        def start_gather(j, b):
            for r in range(RPS):
                pltpu.async_copy(
                    table_hbm.at[idx_v.at[j * RPS + r]],
                    rows_v.at[b].at[r],
                    gsems[b],
                )

        def wait_gather(b):
            for _ in range(RPS):
                pltpu.make_async_copy(
                    table_hbm.at[idx_v.at[0]], rows_v.at[b].at[0], gsems[b]
                ).wait()

        def start_write(j, b):
            pltpu.async_copy(
                rows_v.at[b],
                out_hbm.at[pl.ds(out_base + j * RPS, RPS)],
                wsems[b],
            )

        def wait_write(b):
            pltpu.make_async_copy(
                rows_v.at[b], out_hbm.at[pl.ds(0, RPS)], wsems[b]
            ).wait()

        for b in range(NB):
            start_gather(b, b)

        def body(i, _):
            for b in range(NB):
                wait_gather(b)
                start_write(i * NB + b, b)
            for b in range(NB):
                wait_write(b)
                start_gather(i * NB + b + NB, b)
            return ()

        lax.fori_loop(0, n_loop, body, (), unroll=False)

        for b in range(NB):
            j = NB * n_loop + b
            wait_gather(b)
            start_write(j, b)
        for b in range(n_tail - NB):
            j = NB * n_loop + NB + b
            wait_write(b)
            start_gather(j, b)
            wait_gather(b)
            start_write(j, b)
        for b in range(NB):
            wait_write(b)

    return gather_kernel


def kernel(inputs, embedding):
    B = inputs.size
    idx = inputs.reshape(-1).astype(jnp.int32)
    idx = idx.reshape(NW, B // (NW * IDX_W), IDX_W)
    out = _make_gather(B)(embedding, idx)
    return out.reshape(*inputs.shape, EMBED_DIM)


# speedup vs baseline: 1.0194x; 1.0194x over previous
import functools

import jax
import jax.numpy as jnp
from jax import lax
from jax.experimental import pallas as pl
from jax.experimental.pallas import tpu as pltpu
from jax.experimental.pallas import tpu_sc as plsc

VOCAB_SIZE = 100000
EMBED_DIM = 128

NC = 2
NS = 16
NW = NC * NS

IDX_W = 128
RPS = 1
NB = 6


def _make_gather(B):
    assert B % (NW * IDX_W) == 0
    n_rows = B // (NW * IDX_W)
    assert n_rows % RPS == 0
    n_super = n_rows // RPS
    n_loop = (n_super - NB) // NB
    n_tail = n_super - NB * n_loop
    assert n_loop >= 1 and NB <= n_tail <= 2 * NB

    mesh = plsc.VectorSubcoreMesh(core_axis_name="c", subcore_axis_name="s")

    @functools.partial(
        pl.kernel,
        mesh=mesh,
        out_type=jax.ShapeDtypeStruct((B // IDX_W, IDX_W, EMBED_DIM), jnp.float32),
        scratch_types=[
            pltpu.VMEM((n_rows, IDX_W), jnp.int32),
            pltpu.VMEM((NB, RPS, IDX_W, EMBED_DIM), jnp.float32),
            [pltpu.SemaphoreType.DMA] * NB,
            [pltpu.SemaphoreType.DMA] * NB,
        ],
    )
    def gather_kernel(table_hbm, idx_hbm, out_hbm, idx_v, rows_v, gsems, wsems):
        wid = lax.axis_index("s") * NC + lax.axis_index("c")
        out_base = wid * n_rows

        pltpu.sync_copy(idx_hbm.at[wid], idx_v)

        def start_gather(j, b):
            for r in range(RPS):
                pltpu.async_copy(
                    table_hbm.at[idx_v.at[j * RPS + r]],
                    rows_v.at[b].at[r],
                    gsems[b],
                )

        def wait_gather(b):
            for _ in range(RPS):
                pltpu.make_async_copy(
                    table_hbm.at[idx_v.at[0]], rows_v.at[b].at[0], gsems[b]
                ).wait()

        def start_write(j, b):
            pltpu.async_copy(
                rows_v.at[b],
                out_hbm.at[pl.ds(out_base + j * RPS, RPS)],
                wsems[b],
            )

        def wait_write(b):
            pltpu.make_async_copy(
                rows_v.at[b], out_hbm.at[pl.ds(0, RPS)], wsems[b]
            ).wait()

        for b in range(NB):
            start_gather(b, b)

        def body(i, _):
            for b in range(NB):
                wait_gather(b)
                start_write(i * NB + b, b)
            for b in range(NB):
                wait_write(b)
                start_gather(i * NB + b + NB, b)
            return ()

        lax.fori_loop(0, n_loop, body, (), unroll=False)

        for b in range(NB):
            j = NB * n_loop + b
            wait_gather(b)
            start_write(j, b)
        for b in range(n_tail - NB):
            j = NB * n_loop + NB + b
            wait_write(b)
            start_gather(j, b)
            wait_gather(b)
            start_write(j, b)
        for b in range(NB):
            wait_write(b)

    return gather_kernel


def kernel(inputs, embedding):
    B = inputs.size
    idx = inputs.reshape(-1).astype(jnp.int32)
    idx = idx.reshape(NW, B // (NW * IDX_W), IDX_W)
    out = _make_gather(B)(embedding, idx)
    return out.reshape(*inputs.shape, EMBED_DIM)


# speedup vs baseline: 1.7684x; 1.7348x over previous
import functools

import jax
import jax.numpy as jnp
from jax import lax
from jax.experimental import pallas as pl
from jax.experimental.pallas import tpu as pltpu
from jax.experimental.pallas import tpu_sc as plsc

VOCAB_SIZE = 100000
EMBED_DIM = 128

NC = 2
NS = 16
NW = NC * NS

SPC = 4
NB = 3


def _make_gather(n_seq, seq_len):
    assert n_seq % (NW * SPC) == 0
    seq_per_w = n_seq // NW
    n_chunk = seq_per_w // SPC
    n_loop = (n_chunk - NB) // NB
    n_tail = n_chunk - NB * n_loop
    assert n_loop >= 1 and NB <= n_tail <= 2 * NB

    mesh = plsc.VectorSubcoreMesh(core_axis_name="c", subcore_axis_name="s")

    @functools.partial(
        pl.kernel,
        mesh=mesh,
        out_type=jax.ShapeDtypeStruct((n_seq, seq_len, EMBED_DIM), jnp.float32),
        scratch_types=[
            pltpu.VMEM((seq_per_w, seq_len), jnp.int32),
            pltpu.VMEM((NB, SPC * seq_len, EMBED_DIM), jnp.float32),
            [pltpu.SemaphoreType.DMA] * NB,
            [pltpu.SemaphoreType.DMA] * NB,
        ],
    )
    def gather_kernel(table_hbm, idx_hbm, out_hbm, idx_v, rows_v, gsems, wsems):
        wid = lax.axis_index("s") * NC + lax.axis_index("c")
        seq_base = wid * seq_per_w

        pltpu.sync_copy(idx_hbm.at[wid], idx_v)

        def start_gather(j, b):
            for r in range(SPC):
                pltpu.async_copy(
                    table_hbm.at[idx_v.at[j * SPC + r]],
                    rows_v.at[b].at[pl.ds(r * seq_len, seq_len)],
                    gsems[b],
                )

        def wait_gather(b):
            for _ in range(SPC):
                pltpu.make_async_copy(
                    table_hbm.at[idx_v.at[0]],
                    rows_v.at[b].at[pl.ds(0, seq_len)],
                    gsems[b],
                ).wait()

        def start_write(j, b):
            for r in range(SPC):
                pltpu.async_copy(
                    rows_v.at[b].at[pl.ds(r * seq_len, seq_len)],
                    out_hbm.at[seq_base + j * SPC + r],
                    wsems[b],
                )

        def wait_write(b):
            for _ in range(SPC):
                pltpu.make_async_copy(
                    rows_v.at[b].at[pl.ds(0, seq_len)],
                    out_hbm.at[0],
                    wsems[b],
                ).wait()

        for b in range(NB):
            start_gather(b, b)

        def body(i, _):
            for b in range(NB):
                wait_gather(b)
                start_write(i * NB + b, b)
            for b in range(NB):
                wait_write(b)
                start_gather(i * NB + b + NB, b)
            return ()

        lax.fori_loop(0, n_loop, body, (), unroll=False)

        for b in range(NB):
            j = NB * n_loop + b
            wait_gather(b)
            start_write(j, b)
        for b in range(n_tail - NB):
            j = NB * n_loop + NB + b
            wait_write(b)
            start_gather(j, b)
            wait_gather(b)
            start_write(j, b)
        for b in range(NB):
            wait_write(b)

    return gather_kernel


def kernel(inputs, embedding):
    n_seq, seq_len = inputs.shape
    idx = inputs.astype(jnp.int32).reshape(NW, n_seq // NW, seq_len)
    return _make_gather(n_seq, seq_len)(embedding, idx)
